# baseline (speedup 1.0000x reference)
import functools

import jax
import jax.numpy as jnp
from jax import lax
from jax.experimental import pallas as pl
from jax.experimental.pallas import tpu as pltpu
from jax.experimental.pallas import tpu_sc as plsc

B = 16384
V = 100000
D = 128

_NC = 2
_NS = 16
_NW = _NC * _NS
_BPW = B // _NW


def _sc_gather3(t0, t1, t2, i0, i1, i2):
    mesh = plsc.VectorSubcoreMesh(
        core_axis_name="c", subcore_axis_name="s",
        num_cores=_NC, num_subcores=_NS)

    sub = 2
    ch = _BPW // sub

    @functools.partial(
        pl.kernel,
        out_type=(
            jax.ShapeDtypeStruct((B, D), jnp.float32),
            jax.ShapeDtypeStruct((B, D), jnp.float32),
            jax.ShapeDtypeStruct((B, D), jnp.float32),
        ),
        mesh=mesh,
        scratch_types=[
            pltpu.VMEM((ch,), jnp.int32),
            pltpu.VMEM((ch, D), jnp.float32),
            pltpu.VMEM((ch, D), jnp.float32),
            pltpu.SemaphoreType.DMA,
            pltpu.SemaphoreType.DMA,
            pltpu.SemaphoreType.DMA,
        ],
    )
    def gather_kernel(t0_h, t1_h, t2_h, i0_h, i1_h, i2_h,
                      o0_h, o1_h, o2_h, idx_v, buf0, buf1,
                      sem_g, sem_w0, sem_w1):
        wid = lax.axis_index("s") * _NC + lax.axis_index("c")
        base = wid * _BPW
        tasks = [(tab, idx, out, h * ch)
                 for (tab, idx, out) in ((t0_h, i0_h, o0_h),
                                         (t1_h, i1_h, o1_h),
                                         (t2_h, i2_h, o2_h))
                 for h in range(sub)]
        bufs = (buf0, buf1)
        sems_w = (sem_w0, sem_w1)
        n = len(tasks)
        tab, idx, out, off = tasks[0]
        pltpu.sync_copy(idx.at[pl.ds(base + off, ch)], idx_v)
        g = pltpu.async_copy(tab.at[idx_v], bufs[0], sem_g)
        writes = [None, None]
        for t in range(n):
            b = t % 2
            g.wait()
            tab, idx, out, off = tasks[t]
            writes[b] = pltpu.async_copy(
                bufs[b], out.at[pl.ds(base + off, ch)], sems_w[b])
            if t + 1 < n:
                nb = (t + 1) % 2
                if writes[nb] is not None:
                    writes[nb].wait()
                tab2, idx2, _, off2 = tasks[t + 1]
                pltpu.sync_copy(idx2.at[pl.ds(base + off2, ch)], idx_v)
                g = pltpu.async_copy(tab2.at[idx_v], bufs[nb], sem_g)
        writes[0].wait()
        writes[1].wait()

    return gather_kernel(t0, t1, t2, i0, i1, i2)


_BS = 8192


def _tc_body(e0_r, e1_r, e2_r, w0_r, w1_r, w2_r, b_r, out_r):
    acc = jnp.dot(e0_r[...], w0_r[...], preferred_element_type=jnp.float32)
    acc += jnp.dot(e1_r[...], w1_r[...], preferred_element_type=jnp.float32)
    acc += jnp.dot(e2_r[...], w2_r[...], preferred_element_type=jnp.float32)
    out_r[...] = acc + b_r[...]


def _tc_matmul(e0, e1, e2, w0, w1, w2, bsum):
    eb = pl.BlockSpec((_BS, D), lambda i: (i, 0))
    wb = pl.BlockSpec((D, D), lambda i: (0, 0))
    bb = pl.BlockSpec((1, D), lambda i: (0, 0))
    return pl.pallas_call(
        _tc_body,
        grid=(B // _BS,),
        in_specs=[eb, eb, eb, wb, wb, wb, bb],
        out_specs=pl.BlockSpec((_BS, D), lambda i: (i, 0)),
        out_shape=jax.ShapeDtypeStruct((B, D), jnp.float32),
        compiler_params=pltpu.CompilerParams(
            dimension_semantics=("arbitrary",)),
    )(e0, e1, e2, w0, w1, w2, bsum)


def kernel(Target, Type, Location, T_Target, T_Type, T_Location,
           W0, b0, W1, b1, W2, b2):
    i0 = Target.astype(jnp.int32)
    i1 = Type.astype(jnp.int32)
    i2 = Location.astype(jnp.int32)
    e0, e1, e2 = _sc_gather3(T_Target, T_Type, T_Location, i0, i1, i2)
    bsum = (b0 + b1 + b2).reshape(1, D)
    return _tc_matmul(e0, e1, e2, W0, W1, W2, bsum)


# speedup vs baseline: 1.0706x; 1.0706x over previous
import functools

import jax
import jax.numpy as jnp
from jax import lax
from jax.experimental import pallas as pl
from jax.experimental.pallas import tpu as pltpu
from jax.experimental.pallas import tpu_sc as plsc

B = 16384
V = 100000
D = 128

_NC = 2
_NS = 16
_NW = _NC * _NS
_BPW = B // _NW


def _sc_gather3(t0, t1, t2, i0, i1, i2):
    mesh = plsc.VectorSubcoreMesh(
        core_axis_name="c", subcore_axis_name="s",
        num_cores=_NC, num_subcores=_NS)

    sub = 2
    ch = _BPW // sub
    nbuf = 3
    lookahead = 2

    @functools.partial(
        pl.kernel,
        out_type=(
            jax.ShapeDtypeStruct((B, D), jnp.float32),
            jax.ShapeDtypeStruct((B, D), jnp.float32),
            jax.ShapeDtypeStruct((B, D), jnp.float32),
        ),
        mesh=mesh,
        scratch_types=(
            [pltpu.VMEM((ch,), jnp.int32) for _ in range(nbuf)]
            + [pltpu.VMEM((ch, D), jnp.float32) for _ in range(nbuf)]
            + [pltpu.SemaphoreType.DMA for _ in range(2 * nbuf)]
        ),
    )
    def gather_kernel(t0_h, t1_h, t2_h, i0_h, i1_h, i2_h,
                      o0_h, o1_h, o2_h, *scr):
        idxs = scr[:nbuf]
        bufs = scr[nbuf:2 * nbuf]
        sems_g = scr[2 * nbuf:2 * nbuf + nbuf]
        sems_w = scr[2 * nbuf + nbuf:]
        wid = lax.axis_index("s") * _NC + lax.axis_index("c")
        base = wid * _BPW
        tasks = [(tab, idx, out, h * ch)
                 for (tab, idx, out) in ((t0_h, i0_h, o0_h),
                                         (t1_h, i1_h, o1_h),
                                         (t2_h, i2_h, o2_h))
                 for h in range(sub)]
        n = len(tasks)
        gath = [None] * nbuf
        writes = [None] * nbuf
        for t in range(n + lookahead):
            if t < n:
                slot = t % nbuf
                if writes[slot] is not None:
                    writes[slot].wait()
                    writes[slot] = None
                tab, idx, out, off = tasks[t]
                pltpu.sync_copy(idx.at[pl.ds(base + off, ch)], idxs[slot])
                gath[slot] = pltpu.async_copy(
                    tab.at[idxs[slot]], bufs[slot], sems_g[slot])
            if t >= lookahead:
                u = t - lookahead
                slot = u % nbuf
                gath[slot].wait()
                _, _, out, off = tasks[u]
                writes[slot] = pltpu.async_copy(
                    bufs[slot], out.at[pl.ds(base + off, ch)], sems_w[slot])
        for w in writes:
            if w is not None:
                w.wait()

    return gather_kernel(t0, t1, t2, i0, i1, i2)


_BS = 8192


def _tc_body(e0_r, e1_r, e2_r, w0_r, w1_r, w2_r, b_r, out_r):
    acc = jnp.dot(e0_r[...], w0_r[...], preferred_element_type=jnp.float32)
    acc += jnp.dot(e1_r[...], w1_r[...], preferred_element_type=jnp.float32)
    acc += jnp.dot(e2_r[...], w2_r[...], preferred_element_type=jnp.float32)
    out_r[...] = acc + b_r[...]


def _tc_matmul(e0, e1, e2, w0, w1, w2, bsum):
    eb = pl.BlockSpec((_BS, D), lambda i: (i, 0))
    wb = pl.BlockSpec((D, D), lambda i: (0, 0))
    bb = pl.BlockSpec((1, D), lambda i: (0, 0))
    return pl.pallas_call(
        _tc_body,
        grid=(B // _BS,),
        in_specs=[eb, eb, eb, wb, wb, wb, bb],
        out_specs=pl.BlockSpec((_BS, D), lambda i: (i, 0)),
        out_shape=jax.ShapeDtypeStruct((B, D), jnp.float32),
        compiler_params=pltpu.CompilerParams(
            dimension_semantics=("arbitrary",)),
    )(e0, e1, e2, w0, w1, w2, bsum)


def kernel(Target, Type, Location, T_Target, T_Type, T_Location,
           W0, b0, W1, b1, W2, b2):
    i0 = Target.astype(jnp.int32)
    i1 = Type.astype(jnp.int32)
    i2 = Location.astype(jnp.int32)
    e0, e1, e2 = _sc_gather3(T_Target, T_Type, T_Location, i0, i1, i2)
    bsum = (b0 + b1 + b2).reshape(1, D)
    return _tc_matmul(e0, e1, e2, W0, W1, W2, bsum)
